# baseline (speedup 1.0000x reference)
import functools

import jax
import jax.numpy as jnp
from jax import lax
from jax.experimental import pallas as pl
from jax.experimental.pallas import tpu as pltpu
from jax.experimental.pallas import tpu_sc as plsc

_V, _E, _H, _O = 1000000, 64, 128, 5
_B, _L = 16384, 50
_OP = 16
_BLK = 8000
_BLK8 = _BLK // 8

_NC, _NS = 2, 16
_NW = _NC * _NS
_BPW = _B // _NW
_IPW = _BPW * _L
_CB = 32
_NCH = _BPW // _CB
_ICH = _CB * _L
_G = 80
_NG = _ICH // _G


def _mlp_body(x_ref, w1_ref, b1_ref, w2_ref, b2_ref, w3c_ref, b3r_ref, o_ref):
    bm = lambda a, b: jnp.dot(a.astype(jnp.bfloat16), b.astype(jnp.bfloat16),
                              preferred_element_type=jnp.float32)
    hs = []
    for q in range(4):
        h = jnp.tanh(bm(x_ref[q], w1_ref[...]) + b1_ref[...])
        h = jnp.tanh(bm(h, w2_ref[...]) + b2_ref[...])
        hs.append(h)
    hcat = jnp.concatenate(hs, axis=1)
    o_ref[...] = (bm(hcat, w3c_ref[...]) + b3r_ref[...]).reshape(_BLK8 * 128)


def _precompute_logits(table_r, W1, b1, W2, b2, W3c, b3row):
    return pl.pallas_call(
        _mlp_body,
        grid=(_V // _BLK,),
        in_specs=[
            pl.BlockSpec((4, _BLK8, 2 * _E), lambda i: (0, i, 0)),
            pl.BlockSpec((2 * _E, 2 * _H), lambda i: (0, 0)),
            pl.BlockSpec((1, 2 * _H), lambda i: (0, 0)),
            pl.BlockSpec((2 * _H, 2 * _H), lambda i: (0, 0)),
            pl.BlockSpec((1, 2 * _H), lambda i: (0, 0)),
            pl.BlockSpec((8 * _H, 128), lambda i: (0, 0)),
            pl.BlockSpec((1, 128), lambda i: (0, 0)),
        ],
        out_specs=pl.BlockSpec((_BLK8 * 128,), lambda i: (i,)),
        out_shape=jax.ShapeDtypeStruct((_V * _OP,), jnp.float32),
    )(table_r, W1, b1, W2, b2, W3c, b3row)


def _sc_gather_sum(p, idx_flat):
    mesh = plsc.VectorSubcoreMesh(core_axis_name="c", subcore_axis_name="s",
                                  num_cores=_NC, num_subcores=_NS)

    @functools.partial(
        pl.kernel,
        mesh=mesh,
        compiler_params=pltpu.CompilerParams(use_tc_tiling_on_sc=False),
        out_type=jax.ShapeDtypeStruct((_B, _OP), jnp.float32),
        scratch_types=[
            pltpu.VMEM((_IPW,), jnp.int32),
            pltpu.VMEM((2 * _ICH, _OP), jnp.float32),
            pltpu.VMEM((_CB, _OP), jnp.float32),
            pltpu.SemaphoreType.DMA,
            pltpu.SemaphoreType.DMA,
        ],
    )
    def k(p_hbm, idx_hbm, out_hbm, idx_v, rows_v, out_v, sem0, sem1):
        wid = lax.axis_index("s") * _NC + lax.axis_index("c")
        pltpu.sync_copy(
            idx_hbm.at[pl.ds(pl.multiple_of(wid * _IPW, _IPW), _IPW)], idx_v)
        sems = (sem0, sem1)

        def issue(ch, half):
            cbase = pl.multiple_of(ch * _ICH, _ICH)
            for g in range(_NG):
                pltpu.async_copy(
                    p_hbm.at[idx_v.at[pl.ds(cbase + g * _G, _G)]],
                    rows_v.at[pl.ds(half * _ICH + g * _G, _G)], sems[half])

        def drain(half):
            pltpu.make_async_copy(
                p_hbm.at[pl.ds(0, _ICH)],
                rows_v.at[pl.ds(half * _ICH, _ICH)], sems[half]).wait()

        def accum(ch, half):
            def row(r, c2):
                base = half * _ICH + r * _L
                vals = [rows_v[base + l] for l in range(_L)]
                while len(vals) > 1:
                    nxt = [vals[i] + vals[i + 1]
                           for i in range(0, len(vals) - 1, 2)]
                    if len(vals) % 2:
                        nxt.append(vals[-1])
                    vals = nxt
                out_v[r] = vals[0]
                return c2

            lax.fori_loop(0, _CB, row, 0)
            obase = pl.multiple_of(wid * _BPW + ch * _CB, _CB)
            pltpu.sync_copy(out_v, out_hbm.at[pl.ds(obase, _CB)])

        issue(0, 0)

        def body(i, carry):
            ch0 = 2 * i
            issue(ch0 + 1, 1)
            drain(0)
            accum(ch0, 0)

            @pl.when(i < _NCH // 2 - 1)
            def _():
                issue(ch0 + 2, 0)

            drain(1)
            accum(ch0 + 1, 1)
            return carry

        lax.fori_loop(0, _NCH // 2, body, 0)

    return k(p, idx_flat)


def kernel(inputs, table, W1, b1, W2, b2, W3, b3):
    W3p = jnp.pad(W3, ((0, 0), (0, _OP - _O)))
    zEH = jnp.zeros((_E, _H), jnp.float32)
    zHH = jnp.zeros((_H, _H), jnp.float32)
    W1p = jnp.block([[W1, zEH], [zEH, W1]])
    W2p = jnp.block([[W2, zHH], [zHH, W2]])
    b1p = jnp.tile(b1, 2).reshape(1, 2 * _H)
    b2p = jnp.tile(b2, 2).reshape(1, 2 * _H)
    W3c = jnp.zeros((8 * _H, 128), jnp.float32)
    for q in range(4):
        for j in range(2):
            r0, c0 = 256 * q + 128 * j, 16 * (2 * q + j)
            W3c = W3c.at[r0:r0 + _H, c0:c0 + _OP].set(W3p)
    b3row = jnp.tile(jnp.pad(b3, (0, _OP - _O)), 8).reshape(1, 128)
    p2 = _precompute_logits(table.reshape(4, _V // 8, 2 * _E), W1p,
                            b1p, W2p, b2p, W3c, b3row)
    idx = inputs.reshape(_B * _L)
    q = ((idx >= (_V // 4)).astype(jnp.int32)
         + (idx >= (_V // 2)).astype(jnp.int32)
         + (idx >= (3 * _V // 4)).astype(jnp.int32))
    m = idx - q * (_V // 4)
    idx2 = (m >> 1) * 8 + q * 2 + (m & 1)
    out = _sc_gather_sum(p2.reshape(_V, _OP), idx2)
    return out[:, :_O]


# speedup vs baseline: 1.6151x; 1.6151x over previous
import functools

import jax
import jax.numpy as jnp
from jax import lax
from jax.experimental import pallas as pl
from jax.experimental.pallas import tpu as pltpu
from jax.experimental.pallas import tpu_sc as plsc

_V, _E, _H, _O = 1000000, 64, 128, 5
_B, _L = 16384, 50
_OP = 16
_BLK = 8000
_BLK8 = _BLK // 8

_NC, _NS = 2, 16
_NW = _NC * _NS
_BPW = _B // _NW
_IPW = _BPW * _L
_CB = 32
_NCH = _BPW // _CB
_ICH = _CB * _L
_G = 80
_NG = _ICH // _G


def _mlp_body(x_ref, w1_ref, b1_ref, w2_ref, b2_ref, w3c_ref, b3r_ref, o_ref):
    bm = lambda a, b: jnp.dot(a.astype(jnp.bfloat16), b.astype(jnp.bfloat16),
                              preferred_element_type=jnp.float32)
    hs = []
    for q in range(4):
        xp = jnp.concatenate([x_ref[2 * q], x_ref[2 * q + 1]], axis=1)
        h = jnp.tanh(bm(xp, w1_ref[...]) + b1_ref[...])
        h = jnp.tanh(bm(h, w2_ref[...]) + b2_ref[...])
        hs.append(h)
    hcat = jnp.concatenate(hs, axis=1)
    o_ref[...] = (bm(hcat, w3c_ref[...]) + b3r_ref[...]).reshape(_BLK8 * 128)


def _precompute_logits(table_r, W1, b1, W2, b2, W3c, b3row):
    return pl.pallas_call(
        _mlp_body,
        grid=(_V // _BLK,),
        in_specs=[
            pl.BlockSpec((8, _BLK8, _E), lambda i: (0, i, 0)),
            pl.BlockSpec((2 * _E, 2 * _H), lambda i: (0, 0)),
            pl.BlockSpec((1, 2 * _H), lambda i: (0, 0)),
            pl.BlockSpec((2 * _H, 2 * _H), lambda i: (0, 0)),
            pl.BlockSpec((1, 2 * _H), lambda i: (0, 0)),
            pl.BlockSpec((8 * _H, 128), lambda i: (0, 0)),
            pl.BlockSpec((1, 128), lambda i: (0, 0)),
        ],
        out_specs=pl.BlockSpec((_BLK8 * 128,), lambda i: (i,)),
        out_shape=jax.ShapeDtypeStruct((_V * _OP,), jnp.float32),
    )(table_r, W1, b1, W2, b2, W3c, b3row)


def _sc_gather_sum(p, idx_flat):
    mesh = plsc.VectorSubcoreMesh(core_axis_name="c", subcore_axis_name="s",
                                  num_cores=_NC, num_subcores=_NS)

    @functools.partial(
        pl.kernel,
        mesh=mesh,
        compiler_params=pltpu.CompilerParams(use_tc_tiling_on_sc=False),
        out_type=jax.ShapeDtypeStruct((_B, _OP), jnp.float32),
        scratch_types=[
            pltpu.VMEM((_IPW,), jnp.int32),
            pltpu.VMEM((2 * _ICH, _OP), jnp.float32),
            pltpu.VMEM((_CB, _OP), jnp.float32),
            pltpu.SemaphoreType.DMA,
            pltpu.SemaphoreType.DMA,
        ],
    )
    def k(p_hbm, idx_hbm, out_hbm, idx_v, rows_v, out_v, sem0, sem1):
        wid = lax.axis_index("s") * _NC + lax.axis_index("c")
        pltpu.sync_copy(
            idx_hbm.at[pl.ds(pl.multiple_of(wid * _IPW, _IPW), _IPW)], idx_v)
        sems = (sem0, sem1)

        def issue(ch, half):
            cbase = pl.multiple_of(ch * _ICH, _ICH)
            for g in range(_NG):
                pltpu.async_copy(
                    p_hbm.at[idx_v.at[pl.ds(cbase + g * _G, _G)]],
                    rows_v.at[pl.ds(half * _ICH + g * _G, _G)], sems[half])

        def drain(half):
            pltpu.make_async_copy(
                p_hbm.at[pl.ds(0, _ICH)],
                rows_v.at[pl.ds(half * _ICH, _ICH)], sems[half]).wait()

        def accum(ch, half):
            def row(r, c2):
                base = half * _ICH + r * _L
                vals = [rows_v[base + l] for l in range(_L)]
                while len(vals) > 1:
                    nxt = [vals[i] + vals[i + 1]
                           for i in range(0, len(vals) - 1, 2)]
                    if len(vals) % 2:
                        nxt.append(vals[-1])
                    vals = nxt
                out_v[r] = vals[0]
                return c2

            lax.fori_loop(0, _CB, row, 0)
            obase = pl.multiple_of(wid * _BPW + ch * _CB, _CB)
            pltpu.sync_copy(out_v, out_hbm.at[pl.ds(obase, _CB)])

        issue(0, 0)

        def body(i, carry):
            ch0 = 2 * i
            issue(ch0 + 1, 1)
            drain(0)
            accum(ch0, 0)

            @pl.when(i < _NCH // 2 - 1)
            def _():
                issue(ch0 + 2, 0)

            drain(1)
            accum(ch0 + 1, 1)
            return carry

        lax.fori_loop(0, _NCH // 2, body, 0)

    return k(p, idx_flat)


def kernel(inputs, table, W1, b1, W2, b2, W3, b3):
    W3p = jnp.pad(W3, ((0, 0), (0, _OP - _O)))
    zEH = jnp.zeros((_E, _H), jnp.float32)
    zHH = jnp.zeros((_H, _H), jnp.float32)
    W1p = jnp.block([[W1, zEH], [zEH, W1]])
    W2p = jnp.block([[W2, zHH], [zHH, W2]])
    b1p = jnp.tile(b1, 2).reshape(1, 2 * _H)
    b2p = jnp.tile(b2, 2).reshape(1, 2 * _H)
    W3c = jnp.zeros((8 * _H, 128), jnp.float32)
    for q in range(4):
        for j in range(2):
            r0, c0 = 256 * q + 128 * j, 16 * (2 * q + j)
            W3c = W3c.at[r0:r0 + _H, c0:c0 + _OP].set(W3p)
    b3row = jnp.tile(jnp.pad(b3, (0, _OP - _O)), 8).reshape(1, 128)
    p2 = _precompute_logits(table.reshape(8, _V // 8, _E), W1p,
                            b1p, W2p, b2p, W3c, b3row)
    idx = inputs.reshape(_B * _L)
    a = jnp.zeros_like(idx)
    for kth in range(1, 8):
        a = a + (idx >= kth * (_V // 8)).astype(jnp.int32)
    idx2 = (idx - a * (_V // 8)) * 8 + a
    out = _sc_gather_sum(p2.reshape(_V, _OP), idx2)
    return out[:, :_O]
